# baseline (speedup 1.0000x reference)
import functools

import jax
import jax.numpy as jnp
from jax import lax
from jax.experimental import pallas as pl
from jax.experimental.pallas import tpu as pltpu
from jax.experimental.pallas import tpu_sc as plsc

NC = 2
NS = 16
NW = NC * NS
CH = 128


def _sc_aggregate(nj, n_pad, rows_per_tile, with_counts):
  out_type = [jax.ShapeDtypeStruct((NC, n_pad, 16), jnp.float32)]
  if with_counts:
    out_type.append(jax.ShapeDtypeStruct((NC, n_pad, 16), jnp.float32))

  nb = 8
  nbk = nj // nb
  assert nj % nb == 0

  scratch = [
      pltpu.VMEM((nj, CH), jnp.int32),
      pltpu.VMEM((nj, CH), jnp.int32),
      pltpu.VMEM((2, nb, CH, 16), jnp.float32),
      pltpu.VMEM((CH, 16), jnp.float32),
      pltpu.VMEM_SHARED((n_pad, 16), jnp.float32),
      pltpu.VMEM_SHARED((n_pad, 16), jnp.float32),
      pltpu.SemaphoreType.DMA,
      pltpu.SemaphoreType.DMA,
      pltpu.SemaphoreType.DMA,
  ]

  def body(src3, dst3, table, zeros, ones, *rest):
    if with_counts:
      acc_out, cnt_out = rest[0], rest[1]
      srcb, dstb, rows2, onesb, acc_sh, cnt_sh, gsem, ssem, csem = rest[2:]
    else:
      acc_out = rest[0]
      srcb, dstb, rows2, onesb, acc_sh, cnt_sh, gsem, ssem, csem = rest[1:]
    cid = lax.axis_index("c")
    sid = lax.axis_index("s")
    wid = cid * NS + sid
    rslice = pl.ds(sid * rows_per_tile, rows_per_tile)

    pltpu.sync_copy(zeros, acc_sh.at[rslice])
    if with_counts:
      pltpu.sync_copy(zeros, cnt_sh.at[rslice])
      pltpu.sync_copy(ones, onesb)
    pltpu.sync_copy(src3.at[wid], srcb)
    pltpu.sync_copy(dst3.at[wid], dstb)
    plsc.subcore_barrier()

    for i in range(nb):
      pltpu.async_copy(table.at[srcb.at[i]], rows2.at[0, i], gsem)

    def step(b, carry):
      s = b % 2

      @pl.when(b + 1 < nbk)
      def _():
        for i in range(nb):
          j = (b + 1) * nb + i
          pltpu.async_copy(table.at[srcb.at[j]], rows2.at[(b + 1) % 2, i],
                           gsem)

      for i in range(nb):
        j = b * nb + i
        pltpu.make_async_copy(table.at[srcb.at[j]], rows2.at[s, i],
                              gsem).wait()
        pltpu.async_copy(rows2.at[s, i], acc_sh.at[dstb.at[j]], ssem,
                         add=True)
        if with_counts:
          pltpu.async_copy(onesb, cnt_sh.at[dstb.at[j]], csem, add=True)
      for i in range(nb):
        j = b * nb + i
        pltpu.make_async_copy(rows2.at[s, i], acc_sh.at[dstb.at[j]],
                              ssem).wait()
        if with_counts:
          pltpu.make_async_copy(onesb, cnt_sh.at[dstb.at[j]], csem).wait()
      return carry

    lax.fori_loop(0, nbk, step, 0)
    plsc.subcore_barrier()

    pltpu.sync_copy(acc_sh.at[rslice], acc_out.at[cid, rslice])
    if with_counts:
      pltpu.sync_copy(cnt_sh.at[rslice], cnt_out.at[cid, rslice])

  mesh = plsc.VectorSubcoreMesh(core_axis_name="c", subcore_axis_name="s")
  return pl.kernel(body, out_type=out_type, mesh=mesh, scratch_types=scratch,
                   compiler_params=pltpu.CompilerParams(
                       use_tc_tiling_on_sc=False))


def _mm_kernel(x_ref, w_ref, o_ref):
  o_ref[...] = jnp.dot(x_ref[...], w_ref[...],
                       preferred_element_type=jnp.float32)


def _h_kernel(n, acc_ref, cnt_ref, yz_ref, b0l_ref, g_ref, b_ref, o_ref):
  acc = acc_ref[0, :n, :] + acc_ref[1, :n, :]
  cnt = cnt_ref[0, :n, :] + cnt_ref[1, :n, :]
  agg = acc / jnp.maximum(cnt, 1.0)
  pre = agg + yz_ref[:, 16:32] + b0l_ref[...]
  hr = jnp.maximum(pre, 0.0)
  mu = jnp.mean(hr, axis=1, keepdims=True)
  var = jnp.mean((hr - mu) ** 2, axis=1, keepdims=True)
  o_ref[...] = (hr - mu) / jnp.sqrt(var + 1e-5) * g_ref[...] + b_ref[...]


def _out_kernel(n, acc_ref, cnt_ref, h_ref, w_ref, b_ref, o_ref):
  acc = acc_ref[0, :n, :] + acc_ref[1, :n, :]
  cnt = cnt_ref[0, :n, :] + cnt_ref[1, :n, :]
  agg = acc / jnp.maximum(cnt, 1.0)
  feat = jnp.concatenate([agg, h_ref[...]], axis=1)
  o_ref[...] = jnp.dot(feat, w_ref[...],
                       preferred_element_type=jnp.float32) + b_ref[...]


def kernel(x, edge_index, W0l, b0l, W0r, ln_g, ln_b, W1l, b1l, W1r):
  n, d_in = x.shape
  e = edge_index.shape[1]
  d_hid = W0l.shape[1]
  d_out = W1l.shape[1]

  nj = -(-e // (NW * CH * 8)) * 8
  e_pad = NW * nj * CH
  rows_per_tile = (-(-(n + 8) // NS) + 7) // 8 * 8
  n_pad = NS * rows_per_tile

  src = edge_index[0]
  dst = edge_index[1]
  pad = e_pad - e
  src3 = jnp.concatenate([src, jnp.zeros((pad,), jnp.int32)]).reshape(NW, nj, CH)
  dst3 = jnp.concatenate([dst, jnp.full((pad,), n, jnp.int32)]).reshape(NW, nj, CH)
  zeros = jnp.zeros((rows_per_tile, 16), jnp.float32)
  ones = jnp.ones((CH, 16), jnp.float32)

  wcat0 = jnp.concatenate([W0l, W0r], axis=1)
  y0z0 = pl.pallas_call(
      _mm_kernel,
      out_shape=jax.ShapeDtypeStruct((n, 2 * d_hid), jnp.float32),
  )(x, wcat0)
  y0 = y0z0[:, :d_hid]

  agg_fn = _sc_aggregate(nj, n_pad, rows_per_tile, with_counts=True)
  acc0, cnt = agg_fn(src3, dst3, y0, zeros, ones)

  h = pl.pallas_call(
      functools.partial(_h_kernel, n),
      out_shape=jax.ShapeDtypeStruct((n, d_hid), jnp.float32),
  )(acc0, cnt, y0z0, b0l.reshape(1, -1), ln_g.reshape(1, -1),
    ln_b.reshape(1, -1))

  agg_fn2 = _sc_aggregate(nj, n_pad, rows_per_tile, with_counts=False)
  (acc1,) = agg_fn2(src3, dst3, h, zeros, ones)

  wcat1 = jnp.concatenate([W1l, W1r], axis=0)
  out = pl.pallas_call(
      functools.partial(_out_kernel, n),
      out_shape=jax.ShapeDtypeStruct((n, d_out), jnp.float32),
  )(acc1, cnt, h, wcat1, b1l.reshape(1, -1))
  return out


# speedup vs baseline: 1.1359x; 1.1359x over previous
import functools

import jax
import jax.numpy as jnp
from jax import lax
from jax.experimental import pallas as pl
from jax.experimental.pallas import tpu as pltpu
from jax.experimental.pallas import tpu_sc as plsc

NC = 2
NS = 16
NW = NC * NS
CH = 128


def _sc_aggregate(n, n_pad, rows_per_tile, q0, q1, with_counts):
  out_type = [jax.ShapeDtypeStruct((NC, n_pad, 16), jnp.float32)]
  if with_counts:
    out_type.append(jax.ShapeDtypeStruct((NC, n_pad, 16), jnp.float32))

  assert q0 % CH == 0
  nj0 = q0 // CH
  nj1f = q1 // CH
  tail1 = q1 - nj1f * CH
  nj1 = nj1f + (1 if tail1 else 0)
  njmax = max(nj0, nj1)

  scratch = [
      pltpu.VMEM((njmax * CH,), jnp.int32),
      pltpu.VMEM((njmax, CH), jnp.int32),
      pltpu.VMEM((2, CH, 16), jnp.float32),
      pltpu.VMEM((CH, 16), jnp.float32),
      pltpu.VMEM_SHARED((n_pad, 16), jnp.float32),
      pltpu.VMEM_SHARED((n_pad, 16), jnp.float32),
      pltpu.SemaphoreType.DMA,
      pltpu.SemaphoreType.DMA,
  ]

  def body(ei, table, zeros, ones, zfill, nfill, *rest):
    if with_counts:
      acc_out, cnt_out = rest[0], rest[1]
      srcb, dstb, rows2, onesb, acc_sh, cnt_sh, gsem, isem = rest[2:]
    else:
      acc_out = rest[0]
      srcb, dstb, rows2, onesb, acc_sh, cnt_sh, gsem, isem = rest[1:]
    cid = lax.axis_index("c")
    sid = lax.axis_index("s")
    rslice = pl.ds(sid * rows_per_tile, rows_per_tile)
    on_c0 = cid == 0
    base = jnp.where(on_c0, sid * q0, NS * q0 + sid * q1)
    nj = jnp.where(on_c0, nj0, nj1)

    pltpu.sync_copy(zeros, acc_sh.at[rslice])
    if with_counts:
      pltpu.sync_copy(zeros, cnt_sh.at[rslice])
      pltpu.sync_copy(ones, onesb)

    @pl.when(on_c0)
    def _():
      pltpu.sync_copy(ei.at[0, pl.ds(base, q0)], srcb.at[pl.ds(0, q0)])

    @pl.when(jnp.logical_not(on_c0))
    def _():
      pltpu.sync_copy(ei.at[0, pl.ds(base, q1)], srcb.at[pl.ds(0, q1)])
      if tail1:
        pltpu.sync_copy(zfill.at[pl.ds(0, CH - tail1)],
                        srcb.at[pl.ds(q1, CH - tail1)])

    def stage(j, carry):
      pltpu.async_copy(ei.at[1, pl.ds(base + j * CH, CH)], dstb.at[j], isem)
      return carry

    nfull = jnp.where(on_c0, nj0, nj1f)
    lax.fori_loop(0, nfull, stage, 0)

    @pl.when(jnp.logical_not(on_c0))
    def _():
      if tail1:
        pltpu.sync_copy(nfill, dstb.at[nj1f])
        pltpu.async_copy(ei.at[1, pl.ds(base + nj1f * CH, tail1)],
                         dstb.at[nj1f, pl.ds(0, tail1)], isem)

    def drain(j, carry):
      pltpu.make_async_copy(ei.at[1, pl.ds(base + j * CH, CH)], dstb.at[j],
                            isem).wait()
      return carry

    lax.fori_loop(0, nfull, drain, 0)

    @pl.when(jnp.logical_not(on_c0))
    def _():
      if tail1:
        pltpu.make_async_copy(ei.at[1, pl.ds(base + nj1f * CH, tail1)],
                              dstb.at[nj1f, pl.ds(0, tail1)], isem).wait()

    plsc.subcore_barrier()

    pltpu.async_copy(table.at[srcb.at[pl.ds(0, CH)]], rows2.at[0], gsem)

    def step(j, carry):
      nxt = j + 1

      @pl.when(nxt < nj)
      def _():
        pltpu.async_copy(table.at[srcb.at[pl.ds(nxt * CH, CH)]],
                         rows2.at[nxt % 2], gsem)

      pltpu.make_async_copy(table.at[srcb.at[pl.ds(j * CH, CH)]],
                            rows2.at[j % 2], gsem).wait()
      pltpu.sync_copy(rows2.at[j % 2], acc_sh.at[dstb.at[j]], add=True)
      if with_counts:
        pltpu.sync_copy(onesb, cnt_sh.at[dstb.at[j]], add=True)
      return carry

    lax.fori_loop(0, nj, step, 0)
    plsc.subcore_barrier()

    pltpu.sync_copy(acc_sh.at[rslice], acc_out.at[cid, rslice])
    if with_counts:
      pltpu.sync_copy(cnt_sh.at[rslice], cnt_out.at[cid, rslice])

  mesh = plsc.VectorSubcoreMesh(core_axis_name="c", subcore_axis_name="s")
  return pl.kernel(body, out_type=out_type, mesh=mesh, scratch_types=scratch,
                   compiler_params=pltpu.CompilerParams(
                       use_tc_tiling_on_sc=False))


def _mm_kernel(x_ref, w_ref, o_ref):
  o_ref[...] = jnp.dot(x_ref[...], w_ref[...],
                       preferred_element_type=jnp.float32)


def _h_kernel(n, acc_ref, cnt_ref, yz_ref, b0l_ref, g_ref, b_ref, o_ref):
  acc = acc_ref[0, :n, :] + acc_ref[1, :n, :]
  cnt = cnt_ref[0, :n, :] + cnt_ref[1, :n, :]
  agg = acc / jnp.maximum(cnt, 1.0)
  pre = agg + yz_ref[:, 16:32] + b0l_ref[...]
  hr = jnp.maximum(pre, 0.0)
  mu = jnp.mean(hr, axis=1, keepdims=True)
  var = jnp.mean((hr - mu) ** 2, axis=1, keepdims=True)
  o_ref[...] = (hr - mu) / jnp.sqrt(var + 1e-5) * g_ref[...] + b_ref[...]


def _out_kernel(n, acc_ref, cnt_ref, h_ref, w_ref, b_ref, o_ref):
  acc = acc_ref[0, :n, :] + acc_ref[1, :n, :]
  cnt = cnt_ref[0, :n, :] + cnt_ref[1, :n, :]
  agg = acc / jnp.maximum(cnt, 1.0)
  feat = jnp.concatenate([agg, h_ref[...]], axis=1)
  o_ref[...] = jnp.dot(feat, w_ref[...],
                       preferred_element_type=jnp.float32) + b_ref[...]


def kernel(x, edge_index, W0l, b0l, W0r, ln_g, ln_b, W1l, b1l, W1r):
  n, d_in = x.shape
  e = edge_index.shape[1]
  d_hid = W0l.shape[1]
  d_out = W1l.shape[1]

  rows_per_tile = (-(-(n + 8) // NS) + 7) // 8 * 8
  n_pad = NS * rows_per_tile

  per_tile = e // NS
  q0 = min(int(per_tile * 0.544) // CH * CH, per_tile)
  q1 = per_tile - q0
  assert q0 % 8 == 0 and q1 % 8 == 0 and NS * (q0 + q1) == e

  zeros = jnp.zeros((rows_per_tile, 16), jnp.float32)
  ones = jnp.ones((CH, 16), jnp.float32)
  zfill = jnp.zeros((CH,), jnp.int32)
  nfill = jnp.full((CH,), n, jnp.int32)

  wcat0 = jnp.concatenate([W0l, W0r], axis=1)
  y0z0 = pl.pallas_call(
      _mm_kernel,
      out_shape=jax.ShapeDtypeStruct((n, 2 * d_hid), jnp.float32),
  )(x, wcat0)
  y0 = y0z0[:, :d_hid]

  agg_fn = _sc_aggregate(n, n_pad, rows_per_tile, q0, q1, with_counts=True)
  acc0, cnt = agg_fn(edge_index, y0, zeros, ones, zfill, nfill)

  h = pl.pallas_call(
      functools.partial(_h_kernel, n),
      out_shape=jax.ShapeDtypeStruct((n, d_hid), jnp.float32),
  )(acc0, cnt, y0z0, b0l.reshape(1, -1), ln_g.reshape(1, -1),
    ln_b.reshape(1, -1))

  agg_fn2 = _sc_aggregate(n, n_pad, rows_per_tile, q0, q1, with_counts=False)
  (acc1,) = agg_fn2(edge_index, h, zeros, ones, zfill, nfill)

  wcat1 = jnp.concatenate([W1l, W1r], axis=0)
  out = pl.pallas_call(
      functools.partial(_out_kernel, n),
      out_shape=jax.ShapeDtypeStruct((n, d_out), jnp.float32),
  )(acc1, cnt, h, wcat1, b1l.reshape(1, -1))
  return out


# speedup vs baseline: 1.3466x; 1.1854x over previous
import functools

import jax
import jax.numpy as jnp
from jax import lax
from jax.experimental import pallas as pl
from jax.experimental.pallas import tpu as pltpu
from jax.experimental.pallas import tpu_sc as plsc

NC = 2
NS = 16
NW = NC * NS
CH = 128


def _sc_aggregate(n, n_pad, rows_per_tile, q0, q1, with_counts):
  out_type = [jax.ShapeDtypeStruct((NC, n_pad, 16), jnp.float32)]
  if with_counts:
    out_type.append(jax.ShapeDtypeStruct((NC, n_pad, 16), jnp.float32))

  assert q0 % CH == 0
  nj0 = q0 // CH
  nj1f = q1 // CH
  tail1 = q1 - nj1f * CH
  nj1 = nj1f + (1 if tail1 else 0)
  njmax = max(nj0, nj1)

  scratch = [
      pltpu.VMEM((njmax * CH,), jnp.int32),
      pltpu.VMEM((njmax, CH), jnp.int32),
      pltpu.VMEM((2, CH, 16), jnp.float32),
      pltpu.VMEM((CH, 16), jnp.float32),
      pltpu.VMEM_SHARED((n_pad, 16), jnp.float32),
      pltpu.VMEM_SHARED((n_pad, 16), jnp.float32),
      pltpu.SemaphoreType.DMA,
      pltpu.SemaphoreType.DMA,
  ]

  def body(ei, table, zeros, ones, zfill, nfill, *rest):
    if with_counts:
      acc_out, cnt_out = rest[0], rest[1]
      srcb, dstb, rows2, onesb, acc_sh, cnt_sh, gsem, isem = rest[2:]
    else:
      acc_out = rest[0]
      srcb, dstb, rows2, onesb, acc_sh, cnt_sh, gsem, isem = rest[1:]
    cid = lax.axis_index("c")
    sid = lax.axis_index("s")
    rslice = pl.ds(sid * rows_per_tile, rows_per_tile)
    on_c0 = cid == 0
    base = jnp.where(on_c0, sid * q0, NS * q0 + sid * q1)
    nj = jnp.where(on_c0, nj0, nj1)

    pltpu.sync_copy(zeros, acc_sh.at[rslice])
    if with_counts:
      pltpu.sync_copy(zeros, cnt_sh.at[rslice])
      pltpu.sync_copy(ones, onesb)

    @pl.when(on_c0)
    def _():
      pltpu.sync_copy(ei.at[0, pl.ds(base, q0)], srcb.at[pl.ds(0, q0)])

    @pl.when(jnp.logical_not(on_c0))
    def _():
      pltpu.sync_copy(ei.at[0, pl.ds(base, q1)], srcb.at[pl.ds(0, q1)])
      if tail1:
        pltpu.sync_copy(zfill.at[pl.ds(0, CH - tail1)],
                        srcb.at[pl.ds(q1, CH - tail1)])

    def stage(j, carry):
      pltpu.async_copy(ei.at[1, pl.ds(base + j * CH, CH)], dstb.at[j], isem)
      return carry

    nfull = jnp.where(on_c0, nj0, nj1f)
    lax.fori_loop(0, nfull, stage, 0)

    @pl.when(jnp.logical_not(on_c0))
    def _():
      if tail1:
        pltpu.sync_copy(nfill, dstb.at[nj1f])
        pltpu.async_copy(ei.at[1, pl.ds(base + nj1f * CH, tail1)],
                         dstb.at[nj1f, pl.ds(0, tail1)], isem)

    def drain(j, carry):
      pltpu.make_async_copy(ei.at[1, pl.ds(base + j * CH, CH)], dstb.at[j],
                            isem).wait()
      return carry

    lax.fori_loop(0, nfull, drain, 0)

    @pl.when(jnp.logical_not(on_c0))
    def _():
      if tail1:
        pltpu.make_async_copy(ei.at[1, pl.ds(base + nj1f * CH, tail1)],
                              dstb.at[nj1f, pl.ds(0, tail1)], isem).wait()

    plsc.subcore_barrier()

    pltpu.async_copy(table.at[srcb.at[pl.ds(0, CH)]], rows2.at[0], gsem)

    def step(j, carry):
      nxt = j + 1

      @pl.when(nxt < nj)
      def _():
        pltpu.async_copy(table.at[srcb.at[pl.ds(nxt * CH, CH)]],
                         rows2.at[nxt % 2], gsem)

      pltpu.make_async_copy(table.at[srcb.at[pl.ds(j * CH, CH)]],
                            rows2.at[j % 2], gsem).wait()
      pltpu.sync_copy(rows2.at[j % 2], acc_sh.at[dstb.at[j]], add=True)
      if with_counts:
        pltpu.sync_copy(onesb, cnt_sh.at[dstb.at[j]], add=True)
      return carry

    lax.fori_loop(0, nj, step, 0)
    plsc.subcore_barrier()

    pltpu.sync_copy(acc_sh.at[rslice], acc_out.at[cid, rslice])
    if with_counts:
      pltpu.sync_copy(cnt_sh.at[rslice], cnt_out.at[cid, rslice])

  mesh = plsc.VectorSubcoreMesh(core_axis_name="c", subcore_axis_name="s")
  return pl.kernel(body, out_type=out_type, mesh=mesh, scratch_types=scratch,
                   compiler_params=pltpu.CompilerParams(
                       use_tc_tiling_on_sc=False))


def _mm_kernel(xp_ref, wl_ref, wr_ref, yo_ref, zo_ref):
  xp = xp_ref[...]
  yo_ref[...] = jnp.dot(xp, wl_ref[...], preferred_element_type=jnp.float32)
  zo_ref[...] = jnp.dot(xp, wr_ref[...], preferred_element_type=jnp.float32)


def _h_kernel(m, acc_ref, cnt_ref, z0_ref, b0l_ref, g_ref, b_ref, mdiv_ref,
              o_ref):
  acc = acc_ref[0, :m, :] + acc_ref[1, :m, :]
  cnt = cnt_ref[0, :m, :] + cnt_ref[1, :m, :]
  agg = acc / jnp.maximum(cnt, 1.0)
  pre = agg + z0_ref[...] + b0l_ref[...]
  hr = jnp.maximum(pre, 0.0)
  mdiv = mdiv_ref[...]
  mu = jnp.dot(hr, mdiv, preferred_element_type=jnp.float32)
  d = hr - mu
  var = jnp.dot(d * d, mdiv, preferred_element_type=jnp.float32)
  o_ref[...] = d / jnp.sqrt(var + 1e-5) * g_ref[...] + b_ref[...]


def _out_kernel(m, acc_ref, cnt_ref, h_ref, wl_ref, wr_ref, b_ref, o_ref):
  acc = acc_ref[0, :m, :] + acc_ref[1, :m, :]
  cnt = cnt_ref[0, :m, :] + cnt_ref[1, :m, :]
  agg = acc / jnp.maximum(cnt, 1.0)
  o_ref[...] = (jnp.dot(agg, wl_ref[...], preferred_element_type=jnp.float32)
                + jnp.dot(h_ref[...], wr_ref[...],
                          preferred_element_type=jnp.float32) + b_ref[...])


def kernel(x, edge_index, W0l, b0l, W0r, ln_g, ln_b, W1l, b1l, W1r):
  n, d_in = x.shape
  e = edge_index.shape[1]
  d_hid = W0l.shape[1]
  d_out = W1l.shape[1]

  rows_per_tile = (-(-(n + 8) // NS) + 7) // 8 * 8
  n_pad = NS * rows_per_tile

  per_tile = e // NS
  q0 = min(int(per_tile * 0.525) // CH * CH, per_tile)
  q1 = per_tile - q0
  assert q0 % 8 == 0 and q1 % 8 == 0 and NS * (q0 + q1) == e

  zeros = jnp.zeros((rows_per_tile, 16), jnp.float32)
  ones = jnp.ones((CH, 16), jnp.float32)
  zfill = jnp.zeros((CH,), jnp.int32)
  nfill = jnp.full((CH,), n, jnp.int32)

  pk = 128 // d_hid
  m = n // pk
  m_pad = n_pad // pk
  eye = jnp.eye(pk, dtype=jnp.float32)
  w0l_b = jnp.kron(eye, W0l)
  w0r_b = jnp.kron(eye, W0r)
  w1l_b = jnp.kron(eye, W1l)
  w1r_b = jnp.kron(eye, W1r)
  mdiv = jnp.kron(eye, jnp.full((d_hid, d_hid), 1.0 / d_hid, jnp.float32))
  b0l_t = jnp.tile(b0l, (1, pk))
  g_t = jnp.tile(ln_g, (1, pk))
  bt_t = jnp.tile(ln_b, (1, pk))
  b1l_t = jnp.tile(b1l, (1, pk))

  xp = x.reshape(m, pk * d_in)

  y0p, z0p = pl.pallas_call(
      _mm_kernel,
      out_shape=[jax.ShapeDtypeStruct((m, pk * d_hid), jnp.float32),
                 jax.ShapeDtypeStruct((m, pk * d_hid), jnp.float32)],
  )(xp, w0l_b, w0r_b)

  agg_fn = _sc_aggregate(n, n_pad, rows_per_tile, q0, q1, with_counts=True)
  acc0, cnt = agg_fn(edge_index, y0p.reshape(n, d_hid), zeros, ones, zfill,
                     nfill)
  acc0p = acc0.reshape(NC, m_pad, pk * d_hid)
  cntp = cnt.reshape(NC, m_pad, pk * d_hid)

  hp = pl.pallas_call(
      functools.partial(_h_kernel, m),
      out_shape=jax.ShapeDtypeStruct((m, pk * d_hid), jnp.float32),
  )(acc0p, cntp, z0p, b0l_t, g_t, bt_t, mdiv)

  agg_fn2 = _sc_aggregate(n, n_pad, rows_per_tile, q0, q1, with_counts=False)
  (acc1,) = agg_fn2(edge_index, hp.reshape(n, d_hid), zeros, ones, zfill,
                    nfill)
  acc1p = acc1.reshape(NC, m_pad, pk * d_hid)

  outp = pl.pallas_call(
      functools.partial(_out_kernel, m),
      out_shape=jax.ShapeDtypeStruct((m, pk * d_out), jnp.float32),
  )(acc1p, cntp, hp, w1l_b, w1r_b, b1l_t)
  return outp.reshape(n, d_out)


# speedup vs baseline: 1.4032x; 1.0420x over previous
import functools

import jax
import jax.numpy as jnp
from jax import lax
from jax.experimental import pallas as pl
from jax.experimental.pallas import tpu as pltpu
from jax.experimental.pallas import tpu_sc as plsc

NC = 2
NS = 16
NW = NC * NS
CH = 128


def _sc_aggregate(n, n_pad, rows_per_tile, q0, q1, with_counts):
  out_type = [jax.ShapeDtypeStruct((NC, n_pad, 16), jnp.float32)]
  if with_counts:
    out_type.append(jax.ShapeDtypeStruct((NC, n_pad, 16), jnp.float32))

  assert q0 % CH == 0
  nj0 = q0 // CH
  nj1f = q1 // CH
  tail1 = q1 - nj1f * CH
  nj1 = nj1f + (1 if tail1 else 0)
  njmax = max(nj0, nj1)

  scratch = [
      pltpu.VMEM((njmax * CH,), jnp.int32),
      pltpu.VMEM((njmax, CH), jnp.int32),
      pltpu.VMEM((2, CH, 16), jnp.float32),
      pltpu.VMEM((CH, 16), jnp.float32),
      pltpu.VMEM_SHARED((n_pad, 16), jnp.float32),
      pltpu.VMEM_SHARED((n_pad, 16), jnp.float32),
      pltpu.SemaphoreType.DMA,
      pltpu.SemaphoreType.DMA,
  ]

  def body(ei, table, zeros, ones, zfill, nfill, *rest):
    if with_counts:
      acc_out, cnt_out = rest[0], rest[1]
      srcb, dstb, rows2, onesb, acc_sh, cnt_sh, gsem, isem = rest[2:]
    else:
      acc_out = rest[0]
      srcb, dstb, rows2, onesb, acc_sh, cnt_sh, gsem, isem = rest[1:]
    cid = lax.axis_index("c")
    sid = lax.axis_index("s")
    rslice = pl.ds(sid * rows_per_tile, rows_per_tile)
    on_c0 = cid == 0
    base = jnp.where(on_c0, sid * q0, NS * q0 + sid * q1)
    nj = jnp.where(on_c0, nj0, nj1)

    pltpu.sync_copy(zeros, acc_sh.at[rslice])
    if with_counts:
      pltpu.sync_copy(zeros, cnt_sh.at[rslice])
      pltpu.sync_copy(ones, onesb)

    @pl.when(on_c0)
    def _():
      pltpu.sync_copy(ei.at[0, pl.ds(base, q0)], srcb.at[pl.ds(0, q0)])

    @pl.when(jnp.logical_not(on_c0))
    def _():
      pltpu.sync_copy(ei.at[0, pl.ds(base, q1)], srcb.at[pl.ds(0, q1)])
      if tail1:
        pltpu.sync_copy(zfill.at[pl.ds(0, CH - tail1)],
                        srcb.at[pl.ds(q1, CH - tail1)])

    def stage(j, carry):
      pltpu.async_copy(ei.at[1, pl.ds(base + j * CH, CH)], dstb.at[j], isem)
      return carry

    nfull = jnp.where(on_c0, nj0, nj1f)
    lax.fori_loop(0, nfull, stage, 0)

    @pl.when(jnp.logical_not(on_c0))
    def _():
      if tail1:
        pltpu.sync_copy(nfill, dstb.at[nj1f])
        pltpu.async_copy(ei.at[1, pl.ds(base + nj1f * CH, tail1)],
                         dstb.at[nj1f, pl.ds(0, tail1)], isem)

    def drain(j, carry):
      pltpu.make_async_copy(ei.at[1, pl.ds(base + j * CH, CH)], dstb.at[j],
                            isem).wait()
      return carry

    lax.fori_loop(0, nfull, drain, 0)

    @pl.when(jnp.logical_not(on_c0))
    def _():
      if tail1:
        pltpu.make_async_copy(ei.at[1, pl.ds(base + nj1f * CH, tail1)],
                              dstb.at[nj1f, pl.ds(0, tail1)], isem).wait()

    plsc.subcore_barrier()

    pltpu.async_copy(table.at[srcb.at[pl.ds(0, CH)]], rows2.at[0], gsem)

    def step(j, carry):
      nxt = j + 1

      @pl.when(nxt < nj)
      def _():
        pltpu.async_copy(table.at[srcb.at[pl.ds(nxt * CH, CH)]],
                         rows2.at[nxt % 2], gsem)

      pltpu.make_async_copy(table.at[srcb.at[pl.ds(j * CH, CH)]],
                            rows2.at[j % 2], gsem).wait()
      pltpu.sync_copy(rows2.at[j % 2], acc_sh.at[dstb.at[j]], add=True)
      if with_counts:
        pltpu.sync_copy(onesb, cnt_sh.at[dstb.at[j]], add=True)
      return carry

    lax.fori_loop(0, nj, step, 0)
    plsc.subcore_barrier()

    pltpu.sync_copy(acc_sh.at[rslice], acc_out.at[cid, rslice])
    if with_counts:
      pltpu.sync_copy(cnt_sh.at[rslice], cnt_out.at[cid, rslice])

  mesh = plsc.VectorSubcoreMesh(core_axis_name="c", subcore_axis_name="s")
  return pl.kernel(body, out_type=out_type, mesh=mesh, scratch_types=scratch,
                   compiler_params=pltpu.CompilerParams(
                       use_tc_tiling_on_sc=False))


def _mm_kernel(xp_ref, wl_ref, wr_ref, yo_ref, zo_ref):
  xp = xp_ref[...]
  yo_ref[...] = jnp.dot(xp, wl_ref[...], preferred_element_type=jnp.float32)
  zo_ref[...] = jnp.dot(xp, wr_ref[...], preferred_element_type=jnp.float32)


def _h_kernel(m, acc_ref, cnt_ref, z0_ref, b0l_ref, g_ref, b_ref, mdiv_ref,
              o_ref):
  acc = acc_ref[0, :m, :] + acc_ref[1, :m, :]
  cnt = cnt_ref[0, :m, :] + cnt_ref[1, :m, :]
  agg = acc / jnp.maximum(cnt, 1.0)
  pre = agg + z0_ref[...] + b0l_ref[...]
  hr = jnp.maximum(pre, 0.0)
  mdiv = mdiv_ref[...]
  mu = jnp.dot(hr, mdiv, preferred_element_type=jnp.float32)
  d = hr - mu
  var = jnp.dot(d * d, mdiv, preferred_element_type=jnp.float32)
  o_ref[...] = d / jnp.sqrt(var + 1e-5) * g_ref[...] + b_ref[...]


def _out_kernel(m, acc_ref, cnt_ref, h_ref, wl_ref, wr_ref, b_ref, o_ref):
  acc = acc_ref[0, :m, :] + acc_ref[1, :m, :]
  cnt = cnt_ref[0, :m, :] + cnt_ref[1, :m, :]
  agg = acc / jnp.maximum(cnt, 1.0)
  o = (jnp.dot(agg, wl_ref[...], preferred_element_type=jnp.float32)
       + jnp.dot(h_ref[...], wr_ref[...], preferred_element_type=jnp.float32)
       + b_ref[...])
  o_ref[...] = o.reshape(o_ref.shape)


def kernel(x, edge_index, W0l, b0l, W0r, ln_g, ln_b, W1l, b1l, W1r):
  n, d_in = x.shape
  e = edge_index.shape[1]
  d_hid = W0l.shape[1]
  d_out = W1l.shape[1]

  rows_per_tile = (-(-(n + 8) // NS) + 7) // 8 * 8
  n_pad = NS * rows_per_tile

  per_tile = e // NS
  q0 = min(int(per_tile * 0.521) // CH * CH, per_tile)
  q1 = per_tile - q0
  assert q0 % 8 == 0 and q1 % 8 == 0 and NS * (q0 + q1) == e

  zeros = jnp.zeros((rows_per_tile, 16), jnp.float32)
  ones = jnp.ones((CH, 16), jnp.float32)
  zfill = jnp.zeros((CH,), jnp.int32)
  nfill = jnp.full((CH,), n, jnp.int32)

  pk = 128 // d_hid
  m = n // pk
  m_pad = n_pad // pk
  eye = jnp.eye(pk, dtype=jnp.float32)
  w0l_b = jnp.kron(eye, W0l)
  w0r_b = jnp.kron(eye, W0r)
  w1l_b = jnp.kron(eye, W1l)
  w1r_b = jnp.kron(eye, W1r)
  mdiv = jnp.kron(eye, jnp.full((d_hid, d_hid), 1.0 / d_hid, jnp.float32))
  b0l_t = jnp.tile(b0l, (1, pk))
  g_t = jnp.tile(ln_g, (1, pk))
  bt_t = jnp.tile(ln_b, (1, pk))
  b1l_t = jnp.tile(b1l, (1, pk))

  xp = x.reshape(m, pk * d_in)

  y0p, z0p = pl.pallas_call(
      _mm_kernel,
      out_shape=[jax.ShapeDtypeStruct((m, pk * d_hid), jnp.float32),
                 jax.ShapeDtypeStruct((m, pk * d_hid), jnp.float32)],
  )(xp, w0l_b, w0r_b)

  agg_fn = _sc_aggregate(n, n_pad, rows_per_tile, q0, q1, with_counts=True)
  acc0, cnt = agg_fn(edge_index, y0p.reshape(n, d_hid), zeros, ones, zfill,
                     nfill)
  acc0p = acc0.reshape(NC, m_pad, pk * d_hid)
  cntp = cnt.reshape(NC, m_pad, pk * d_hid)

  hp = pl.pallas_call(
      functools.partial(_h_kernel, m),
      out_shape=jax.ShapeDtypeStruct((m, pk * d_hid), jnp.float32),
  )(acc0p, cntp, z0p, b0l_t, g_t, bt_t, mdiv)

  agg_fn2 = _sc_aggregate(n, n_pad, rows_per_tile, q0, q1, with_counts=False)
  (acc1,) = agg_fn2(edge_index, hp.reshape(n, d_hid), zeros, ones, zfill,
                    nfill)
  acc1p = acc1.reshape(NC, m_pad, pk * d_hid)

  out = pl.pallas_call(
      functools.partial(_out_kernel, m),
      out_shape=jax.ShapeDtypeStruct((n, d_out), jnp.float32),
  )(acc1p, cntp, hp, w1l_b, w1r_b, b1l_t)
  return out


# speedup vs baseline: 2.0031x; 1.4276x over previous
import functools

import jax
import jax.numpy as jnp
from jax import lax
from jax.experimental import pallas as pl
from jax.experimental.pallas import tpu as pltpu
from jax.experimental.pallas import tpu_sc as plsc

NC = 2
NS = 16
NW = NC * NS
CH = 128


def _sc_aggregate(n, n_pad, rows_per_tile, q0, q1, with_counts):
  out_type = [jax.ShapeDtypeStruct((NC, n_pad, 16), jnp.float32)]
  if with_counts:
    out_type.append(jax.ShapeDtypeStruct((NC, n_pad, 16), jnp.float32))

  assert q0 % CH == 0
  nj0 = q0 // CH
  nj1f = q1 // CH
  tail1 = q1 - nj1f * CH
  nj1 = nj1f + (1 if tail1 else 0)
  njmax = max(nj0, nj1)

  scratch = [
      pltpu.VMEM((njmax * CH,), jnp.int32),
      pltpu.VMEM((njmax, CH), jnp.int32),
      pltpu.VMEM((2, CH, 16), jnp.float32),
      pltpu.VMEM((CH, 16), jnp.float32),
      pltpu.VMEM_SHARED((n_pad, 16), jnp.float32),
      pltpu.VMEM_SHARED((n_pad, 16), jnp.float32),
      pltpu.VMEM_SHARED((n_pad, 16), jnp.float32),
      pltpu.SemaphoreType.DMA,
      pltpu.SemaphoreType.DMA,
  ]
  tab_rows = n_pad // NS

  def body(ei, table, zeros, ones, zfill, nfill, *rest):
    if with_counts:
      acc_out, cnt_out = rest[0], rest[1]
      srcb, dstb, rows2, onesb, acc_sh, cnt_sh, tab_sh, gsem, isem = rest[2:]
    else:
      acc_out = rest[0]
      srcb, dstb, rows2, onesb, acc_sh, cnt_sh, tab_sh, gsem, isem = rest[1:]
    cid = lax.axis_index("c")
    sid = lax.axis_index("s")
    rslice = pl.ds(sid * rows_per_tile, rows_per_tile)
    on_c0 = cid == 0
    base = jnp.where(on_c0, sid * q0, NS * q0 + sid * q1)
    nj = jnp.where(on_c0, nj0, nj1)

    pltpu.sync_copy(zeros, acc_sh.at[rslice])
    if with_counts:
      pltpu.sync_copy(zeros, cnt_sh.at[rslice])
      pltpu.sync_copy(ones, onesb)

    @pl.when(on_c0)
    def _():
      pltpu.sync_copy(ei.at[0, pl.ds(base, q0)], srcb.at[pl.ds(0, q0)])

    @pl.when(jnp.logical_not(on_c0))
    def _():
      pltpu.sync_copy(ei.at[0, pl.ds(base, q1)], srcb.at[pl.ds(0, q1)])
      if tail1:
        pltpu.sync_copy(zfill.at[pl.ds(0, CH - tail1)],
                        srcb.at[pl.ds(q1, CH - tail1)])

    def stage(j, carry):
      pltpu.async_copy(ei.at[1, pl.ds(base + j * CH, CH)], dstb.at[j], isem)
      return carry

    nfull = jnp.where(on_c0, nj0, nj1f)
    lax.fori_loop(0, nfull, stage, 0)

    @pl.when(jnp.logical_not(on_c0))
    def _():
      if tail1:
        pltpu.sync_copy(nfill, dstb.at[nj1f])
        pltpu.async_copy(ei.at[1, pl.ds(base + nj1f * CH, tail1)],
                         dstb.at[nj1f, pl.ds(0, tail1)], isem)

    def drain(j, carry):
      pltpu.make_async_copy(ei.at[1, pl.ds(base + j * CH, CH)], dstb.at[j],
                            isem).wait()
      return carry

    lax.fori_loop(0, nfull, drain, 0)

    @pl.when(jnp.logical_not(on_c0))
    def _():
      if tail1:
        pltpu.make_async_copy(ei.at[1, pl.ds(base + nj1f * CH, tail1)],
                              dstb.at[nj1f, pl.ds(0, tail1)], isem).wait()

    last_rows = n - (NS - 1) * tab_rows

    @pl.when(sid < NS - 1)
    def _():
      trs = pl.ds(sid * tab_rows, tab_rows)
      pltpu.sync_copy(table.at[trs], tab_sh.at[trs])

    @pl.when(sid == NS - 1)
    def _():
      trs = pl.ds((NS - 1) * tab_rows, last_rows)
      pltpu.sync_copy(table.at[trs], tab_sh.at[trs])

    plsc.subcore_barrier()

    pltpu.async_copy(tab_sh.at[srcb.at[pl.ds(0, CH)]], rows2.at[0], gsem)

    def step(j, carry):
      nxt = j + 1

      @pl.when(nxt < nj)
      def _():
        pltpu.async_copy(tab_sh.at[srcb.at[pl.ds(nxt * CH, CH)]],
                         rows2.at[nxt % 2], gsem)

      pltpu.make_async_copy(tab_sh.at[srcb.at[pl.ds(j * CH, CH)]],
                            rows2.at[j % 2], gsem).wait()
      pltpu.sync_copy(rows2.at[j % 2], acc_sh.at[dstb.at[j]], add=True)
      if with_counts:
        pltpu.sync_copy(onesb, cnt_sh.at[dstb.at[j]], add=True)
      return carry

    lax.fori_loop(0, nj, step, 0)
    plsc.subcore_barrier()

    pltpu.sync_copy(acc_sh.at[rslice], acc_out.at[cid, rslice])
    if with_counts:
      pltpu.sync_copy(cnt_sh.at[rslice], cnt_out.at[cid, rslice])

  mesh = plsc.VectorSubcoreMesh(core_axis_name="c", subcore_axis_name="s")
  return pl.kernel(body, out_type=out_type, mesh=mesh, scratch_types=scratch,
                   compiler_params=pltpu.CompilerParams(
                       use_tc_tiling_on_sc=False))


def _mm_kernel(xp_ref, wl_ref, wr_ref, yo_ref, zo_ref):
  xp = xp_ref[...]
  yo_ref[...] = jnp.dot(xp, wl_ref[...], preferred_element_type=jnp.float32)
  zo_ref[...] = jnp.dot(xp, wr_ref[...], preferred_element_type=jnp.float32)


def _h_kernel(m, acc_ref, cnt_ref, z0_ref, b0l_ref, g_ref, b_ref, mdiv_ref,
              o_ref):
  acc = acc_ref[0, :m, :] + acc_ref[1, :m, :]
  cnt = cnt_ref[0, :m, :] + cnt_ref[1, :m, :]
  agg = acc / jnp.maximum(cnt, 1.0)
  pre = agg + z0_ref[...] + b0l_ref[...]
  hr = jnp.maximum(pre, 0.0)
  mdiv = mdiv_ref[...]
  mu = jnp.dot(hr, mdiv, preferred_element_type=jnp.float32)
  d = hr - mu
  var = jnp.dot(d * d, mdiv, preferred_element_type=jnp.float32)
  o_ref[...] = d / jnp.sqrt(var + 1e-5) * g_ref[...] + b_ref[...]


def _out_kernel(m, acc_ref, cnt_ref, h_ref, wl_ref, wr_ref, b_ref, o_ref):
  acc = acc_ref[0, :m, :] + acc_ref[1, :m, :]
  cnt = cnt_ref[0, :m, :] + cnt_ref[1, :m, :]
  agg = acc / jnp.maximum(cnt, 1.0)
  o = (jnp.dot(agg, wl_ref[...], preferred_element_type=jnp.float32)
       + jnp.dot(h_ref[...], wr_ref[...], preferred_element_type=jnp.float32)
       + b_ref[...])
  o_ref[...] = o.reshape(o_ref.shape)


def kernel(x, edge_index, W0l, b0l, W0r, ln_g, ln_b, W1l, b1l, W1r):
  n, d_in = x.shape
  e = edge_index.shape[1]
  d_hid = W0l.shape[1]
  d_out = W1l.shape[1]

  rows_per_tile = (-(-(n + 8) // NS) + 7) // 8 * 8
  n_pad = NS * rows_per_tile

  per_tile = e // NS
  q0 = min(int(per_tile * 0.521) // CH * CH, per_tile)
  q1 = per_tile - q0
  assert q0 % 8 == 0 and q1 % 8 == 0 and NS * (q0 + q1) == e

  zeros = jnp.zeros((rows_per_tile, 16), jnp.float32)
  ones = jnp.ones((CH, 16), jnp.float32)
  zfill = jnp.zeros((CH,), jnp.int32)
  nfill = jnp.full((CH,), n, jnp.int32)

  pk = 128 // d_hid
  m = n // pk
  m_pad = n_pad // pk
  eye = jnp.eye(pk, dtype=jnp.float32)
  w0l_b = jnp.kron(eye, W0l)
  w0r_b = jnp.kron(eye, W0r)
  w1l_b = jnp.kron(eye, W1l)
  w1r_b = jnp.kron(eye, W1r)
  mdiv = jnp.kron(eye, jnp.full((d_hid, d_hid), 1.0 / d_hid, jnp.float32))
  b0l_t = jnp.tile(b0l, (1, pk))
  g_t = jnp.tile(ln_g, (1, pk))
  bt_t = jnp.tile(ln_b, (1, pk))
  b1l_t = jnp.tile(b1l, (1, pk))

  xp = x.reshape(m, pk * d_in)

  y0p, z0p = pl.pallas_call(
      _mm_kernel,
      out_shape=[jax.ShapeDtypeStruct((m, pk * d_hid), jnp.float32),
                 jax.ShapeDtypeStruct((m, pk * d_hid), jnp.float32)],
  )(xp, w0l_b, w0r_b)

  agg_fn = _sc_aggregate(n, n_pad, rows_per_tile, q0, q1, with_counts=True)
  acc0, cnt = agg_fn(edge_index, y0p.reshape(n, d_hid), zeros, ones, zfill,
                     nfill)
  acc0p = acc0.reshape(NC, m_pad, pk * d_hid)
  cntp = cnt.reshape(NC, m_pad, pk * d_hid)

  hp = pl.pallas_call(
      functools.partial(_h_kernel, m),
      out_shape=jax.ShapeDtypeStruct((m, pk * d_hid), jnp.float32),
  )(acc0p, cntp, z0p, b0l_t, g_t, bt_t, mdiv)

  agg_fn2 = _sc_aggregate(n, n_pad, rows_per_tile, q0, q1, with_counts=False)
  (acc1,) = agg_fn2(edge_index, hp.reshape(n, d_hid), zeros, ones, zfill,
                    nfill)
  acc1p = acc1.reshape(NC, m_pad, pk * d_hid)

  out = pl.pallas_call(
      functools.partial(_out_kernel, m),
      out_shape=jax.ShapeDtypeStruct((n, d_out), jnp.float32),
  )(acc1p, cntp, hp, w1l_b, w1r_b, b1l_t)
  return out
